# baseline (speedup 1.0000x reference)
import jax
import jax.numpy as jnp
from jax.experimental import pallas as pl
from jax.experimental.pallas import tpu as pltpu

_MAX_LEN = 200
_EMBED_DIM = 64
_FLAT = _MAX_LEN * _EMBED_DIM
_BB = 256


def _broadcast_body(pe_ref, o_ref):
    o_ref[...] = jnp.broadcast_to(pe_ref[...], o_ref.shape)


def kernel(x, pe_weight):
    batch = x.shape[0]
    pe_flat = pe_weight.reshape(1, _FLAT)
    out = pl.pallas_call(
        _broadcast_body,
        grid=(batch // _BB,),
        in_specs=[pl.BlockSpec((1, _FLAT), lambda i: (0, 0))],
        out_specs=pl.BlockSpec((_BB, _FLAT), lambda i: (i, 0)),
        out_shape=jax.ShapeDtypeStruct((batch, _FLAT), jnp.float32),
        compiler_params=pltpu.CompilerParams(
            dimension_semantics=("parallel",),
        ),
    )(pe_flat)
    return out.reshape(batch, _MAX_LEN, _EMBED_DIM)
